# baseline (speedup 1.0000x reference)
import functools

import jax
import jax.numpy as jnp
from jax import lax
from jax.experimental import pallas as pl
from jax.experimental.pallas import tpu as pltpu
from jax.experimental.pallas import tpu_sc as plsc

EMBED = 64


def kernel(next_id, next_category, next_subcategory, id_table, category_table,
           subcategory_table):
    B = next_id.shape[0]
    next_id = next_id.astype(jnp.int32)
    next_category = next_category.astype(jnp.int32)
    next_subcategory = next_subcategory.astype(jnp.int32)

    info = plsc.get_sparse_core_info()
    nw = info.num_cores * info.num_subcores
    b_per_w = B // nw

    mesh = plsc.VectorSubcoreMesh(core_axis_name="c", subcore_axis_name="s")

    @functools.partial(
        pl.kernel,
        mesh=mesh,
        out_type=[jax.ShapeDtypeStruct((B, EMBED), jnp.float32)] * 3,
        compiler_params=pltpu.CompilerParams(use_tc_tiling_on_sc=False),
        scratch_types=[
            [pltpu.VMEM((b_per_w,), jnp.int32) for _ in range(3)],
            [pltpu.VMEM((b_per_w, EMBED), jnp.float32) for _ in range(3)],
            [pltpu.SemaphoreType.DMA for _ in range(3)],
        ],
    )
    def gather3(id_idx_hbm, cat_idx_hbm, sub_idx_hbm, id_tab, cat_tab,
                sub_tab, out0, out1, out2, idxs, rows, sems):
        wid = lax.axis_index("s") * info.num_cores + lax.axis_index("c")
        base = wid * b_per_w
        tabs = (id_tab, cat_tab, sub_tab)
        outs = (out0, out1, out2)
        for t, idx_hbm in enumerate((id_idx_hbm, cat_idx_hbm, sub_idx_hbm)):
            pltpu.sync_copy(idx_hbm.at[pl.ds(base, b_per_w)], idxs[t])
        copies = [
            pltpu.async_copy(tabs[t].at[idxs[t]], rows[t], sems[t])
            for t in range(3)
        ]
        for t in range(3):
            copies[t].wait()
            pltpu.sync_copy(rows[t], outs[t].at[pl.ds(base, b_per_w)])

    o0, o1, o2 = gather3(next_id, next_category, next_subcategory, id_table,
                         category_table, subcategory_table)
    return jnp.concatenate([o0, o1, o2], axis=1)
